# baseline (speedup 1.0000x reference)
import jax
import jax.numpy as jnp
import numpy as np
from jax import lax
from jax.experimental import pallas as pl
from jax.experimental.pallas import tpu as pltpu
from jax.experimental.pallas import tpu_sc as plsc

N = 160
NUM_SPECIES = 7
RCR = 5.1
RCA = 3.5
ETA_R = 19.7
SHF_R0 = 0.8
SHF_R_STEP = 0.26875
N_SHF_R = 16
ZETA = 14.1
SHF_Z = [0.19634954, 0.58904862, 0.9817477, 1.3744468, 1.7671459, 2.1598449,
         2.552544, 2.9452431]
ETA_A = 12.5
SHF_A = [0.8, 1.1375, 1.475, 1.8125, 2.15, 2.4875, 2.825, 3.1625]
NUM_PAIRS = NUM_SPECIES * (NUM_SPECIES + 1) // 2
N_FEAT = NUM_SPECIES * N_SHF_R + NUM_PAIRS * len(SHF_Z) * len(SHF_A)
PI = float(np.pi)
LN2 = float(np.log(2.0))

_COS_Z = [float(np.cos(np.float32(z))) for z in SHF_Z]
_SIN_Z = [float(np.sin(np.float32(z))) for z in SHF_Z]

NC = 2
NS = 16
NW = NC * NS
CPW = N // NW
NCHUNK = N // 16
CAP = 176


def _sqrt16(x, iters=3):
    ok = x > 1e-12
    xs = jnp.where(ok, x, 1.0)
    i = plsc.bitcast(xs, jnp.int32)
    y = plsc.bitcast(jnp.int32(0x5F3759DF) - lax.shift_right_logical(i, 1),
                     jnp.float32)
    for _ in range(iters):
        y = y * (1.5 - 0.5 * xs * y * y)
    return jnp.where(ok, xs * y, 0.0)


def _fc16(d, rc):
    x = jnp.minimum(d * (PI / (2.0 * rc)), PI / 2.0)
    u = x * x
    c = 1.0 + u * (-1.0 / 2 + u * (1.0 / 24 + u * (-1.0 / 720 + u * (
        1.0 / 40320 + u * (-1.0 / 3628800 + u * (1.0 / 479001600))))))
    return c * c


def _ln16(y):
    i = plsc.bitcast(y, jnp.int32)
    t = i.astype(jnp.float32) * (2.0 ** -23) - 127.0
    f = (i & jnp.int32(0x7FFFFF)).astype(jnp.float32) * (2.0 ** -23)
    return LN2 * (t + 0.346607 * f * (1.0 - f))


def _sc_body(pos_hbm, out_hbm,
             pos_v, nbx, nby, nbz, nbd, nbf, nrd, nrf, acc_v):
    cid = lax.axis_index("c")
    sid = lax.axis_index("s")
    wid = sid * NC + cid
    pltpu.sync_copy(pos_hbm, pos_v)
    lane = lax.iota(jnp.int32, 16)
    lane_f = lane.astype(jnp.float32)
    shfr = SHF_R0 + SHF_R_STEP * lane_f

    def center_body(t, acc_in):
        i = wid * CPW + t
        iv = jnp.full((16,), 3 * i, jnp.int32)
        xi = plsc.load_gather(pos_v, [iv])
        yi = plsc.load_gather(pos_v, [iv + 1])
        zi = plsc.load_gather(pos_v, [iv + 2])

        def counts_body(ci, carry):
            nc, nr = carry
            base = ci * 16
            idx3 = (base + lane) * 3
            xj = plsc.load_gather(pos_v, [idx3])
            yj = plsc.load_gather(pos_v, [idx3 + 1])
            zj = plsc.load_gather(pos_v, [idx3 + 2])
            dx = xj - xi
            dy = yj - yi
            dz = zj - zi
            d = _sqrt16(dx * dx + dy * dy + dz * dz)
            notself = (base + lane) != i
            mr = (d <= RCR) & notself
            frad = jnp.where(mr, 0.25 * _fc16(d, RCR), 0.0)
            incr = plsc.cumsum(mr.astype(jnp.int32))
            posr = nr + incr - 1
            plsc.store_scatter(nrd, [posr], d, mask=mr)
            plsc.store_scatter(nrf, [posr], frad, mask=mr)
            ma = (d <= RCA) & notself
            fca = jnp.where(ma, _fc16(d, RCA), 0.0)
            inca = plsc.cumsum(ma.astype(jnp.int32))
            posa = nc + inca - 1
            plsc.store_scatter(nbx, [posa], xj, mask=ma)
            plsc.store_scatter(nby, [posa], yj, mask=ma)
            plsc.store_scatter(nbz, [posa], zj, mask=ma)
            plsc.store_scatter(nbd, [posa], d, mask=ma)
            plsc.store_scatter(nbf, [posa], fca, mask=ma)
            return nc + jnp.max(inca), nr + jnp.max(incr)

        nc, nr = lax.fori_loop(0, NCHUNK, counts_body,
                               (jnp.int32(0), jnp.int32(0)))

        def r_body(rj, a):
            rv = jnp.full((16,), rj, jnp.int32)
            dv = plsc.load_gather(nrd, [rv])
            fv = plsc.load_gather(nrf, [rv])
            ts = dv - shfr
            return a + fv * jnp.exp(-ETA_R * (ts * ts))

        acc_r = lax.fori_loop(0, nr, r_body, acc_in)

        npair = lax.shift_right_logical(nc * (nc - 1), 1)
        npc = lax.shift_right_logical(npair + 15, 4)

        def p_body(pc, a):
            tt = pc * 16 + lane
            tf = tt.astype(jnp.float32)
            kf = (1.0 + _sqrt16(1.0 + 8.0 * tf)) * 0.5
            kk = kf.astype(jnp.int32)
            tri = lax.shift_right_logical(kk * (kk - 1), 1)
            too_big = tt < tri
            kk = jnp.where(too_big, kk - 1, kk)
            tri = jnp.where(too_big, lax.shift_right_logical(kk * (kk - 1), 1),
                            tri)
            too_small = tt >= tri + kk
            kk = jnp.where(too_small, kk + 1, kk)
            tri = jnp.where(too_small,
                            lax.shift_right_logical(kk * (kk - 1), 1), tri)
            jj = tt - tri
            valid = tt < npair
            jj = jnp.where(valid, jj, 0)
            kk = jnp.where(valid, kk, 1)
            xj = plsc.load_gather(nbx, [jj])
            yj = plsc.load_gather(nby, [jj])
            zj = plsc.load_gather(nbz, [jj])
            dj = plsc.load_gather(nbd, [jj])
            fj = plsc.load_gather(nbf, [jj])
            xk = plsc.load_gather(nbx, [kk])
            yk = plsc.load_gather(nby, [kk])
            zk = plsc.load_gather(nbz, [kk])
            dk = plsc.load_gather(nbd, [kk])
            fk = plsc.load_gather(nbf, [kk])
            fk = jnp.where(valid, fk, 0.0)
            dots = ((xj - xi) * (xk - xi) + (yj - yi) * (yk - yi)
                    + (zj - zi) * (zk - zi))
            denom = jnp.maximum(dj * dk, 1e-10)
            c = jnp.clip(0.95 * dots / denom, -0.95, 0.95)
            s = _sqrt16(jnp.maximum(1.0 - c * c, 0.0), iters=2)
            avg = (dj + dk) * 0.5
            f2 = jnp.zeros((16,), jnp.float32)
            for sa in SHF_A:
                ta = avg - sa
                f2 = f2 + jnp.exp(-ETA_A * (ta * ta))
            f1 = jnp.zeros((16,), jnp.float32)
            for cz, sz in zip(_COS_Z, _SIN_Z):
                y = jnp.maximum((1.0 + c * cz + s * sz) * 0.5, 1e-30)
                y2 = y * y
                y4 = y2 * y2
                y8 = y4 * y4
                y14 = y8 * y4 * y2
                f1 = f1 + y14 * jnp.exp(0.1 * _ln16(y))
            return a + (2.0 * fj * fk) * (f1 * f2)

        return lax.fori_loop(0, npc, p_body, acc_r)

    acc = lax.fori_loop(0, CPW, center_body, jnp.zeros((16,), jnp.float32))
    acc_v[...] = acc
    pltpu.sync_copy(acc_v, out_hbm.at[wid])


@jax.jit
def _aev_mean_sc(positions):
    pos_flat = positions.astype(jnp.float32).reshape(3 * N)
    mesh = plsc.VectorSubcoreMesh(core_axis_name="c", subcore_axis_name="s")
    f32 = jnp.float32
    sck = pl.kernel(
        _sc_body,
        out_type=jax.ShapeDtypeStruct((NW, 16), f32),
        mesh=mesh,
        compiler_params=pltpu.CompilerParams(needs_layout_passes=False),
        scratch_types=[
            pltpu.VMEM((3 * N,), f32),
            pltpu.VMEM((CAP,), f32), pltpu.VMEM((CAP,), f32),
            pltpu.VMEM((CAP,), f32), pltpu.VMEM((CAP,), f32),
            pltpu.VMEM((CAP,), f32),
            pltpu.VMEM((CAP,), f32), pltpu.VMEM((CAP,), f32),
            pltpu.VMEM((16,), f32),
        ],
    )
    out = sck(pos_flat)
    return jnp.sum(out) * (1.0 / (N * N_FEAT))


def kernel(species, positions):
    del species
    return _aev_mean_sc(positions)


# speedup vs baseline: 1.0504x; 1.0504x over previous
import jax
import jax.numpy as jnp
import numpy as np
from jax import lax
from jax.experimental import pallas as pl
from jax.experimental.pallas import tpu as pltpu
from jax.experimental.pallas import tpu_sc as plsc

N = 160
NUM_SPECIES = 7
RCR = 5.1
RCA = 3.5
ETA_R = 19.7
SHF_R0 = 0.8
SHF_R_STEP = 0.26875
N_SHF_R = 16
ZETA = 14.1
SHF_Z = [0.19634954, 0.58904862, 0.9817477, 1.3744468, 1.7671459, 2.1598449,
         2.552544, 2.9452431]
ETA_A = 12.5
SHF_A = [0.8, 1.1375, 1.475, 1.8125, 2.15, 2.4875, 2.825, 3.1625]
NUM_PAIRS = NUM_SPECIES * (NUM_SPECIES + 1) // 2
N_FEAT = NUM_SPECIES * N_SHF_R + NUM_PAIRS * len(SHF_Z) * len(SHF_A)
PI = float(np.pi)
LN2 = float(np.log(2.0))

_COS_Z = [float(np.cos(np.float32(z))) for z in SHF_Z]
_SIN_Z = [float(np.sin(np.float32(z))) for z in SHF_Z]

NC = 2
NS = 16
NW = NC * NS
CPW = N // NW
NCHUNK = N // 16
CAP = 176


def _sqrt16(x, iters=3):
    ok = x > 1e-12
    xs = jnp.where(ok, x, 1.0)
    i = plsc.bitcast(xs, jnp.int32)
    y = plsc.bitcast(jnp.int32(0x5F3759DF) - lax.shift_right_logical(i, 1),
                     jnp.float32)
    for _ in range(iters):
        y = y * (1.5 - 0.5 * xs * y * y)
    return jnp.where(ok, xs * y, 0.0)


def _fc16(d, rc):
    x = jnp.minimum(d * (PI / (2.0 * rc)), PI / 2.0)
    u = x * x
    c = 1.0 + u * (-1.0 / 2 + u * (1.0 / 24 + u * (-1.0 / 720 + u * (
        1.0 / 40320 + u * (-1.0 / 3628800 + u * (1.0 / 479001600))))))
    return c * c


def _ln16(y):
    i = plsc.bitcast(y, jnp.int32)
    t = i.astype(jnp.float32) * (2.0 ** -23) - 127.0
    f = (i & jnp.int32(0x7FFFFF)).astype(jnp.float32) * (2.0 ** -23)
    return LN2 * (t + 0.346607 * f * (1.0 - f))


def _sc_body(pos_hbm, out_hbm,
             pos_v, nbx, nby, nbz, nbd, nbf, nrd, nrf, acc_v):
    cid = lax.axis_index("c")
    sid = lax.axis_index("s")
    wid = sid * NC + cid
    pltpu.sync_copy(pos_hbm, pos_v)
    lane = lax.iota(jnp.int32, 16)
    lane_f = lane.astype(jnp.float32)
    shfr = SHF_R0 + SHF_R_STEP * lane_f

    def center_body(t, acc_in):
        i = wid * CPW + t
        iv = jnp.full((16,), 3 * i, jnp.int32)
        xi = plsc.load_gather(pos_v, [iv])
        yi = plsc.load_gather(pos_v, [iv + 1])
        zi = plsc.load_gather(pos_v, [iv + 2])

        @plsc.parallel_loop(0, NCHUNK, 1, unroll=5,
                            carry=(jnp.int32(0), jnp.int32(0)))
        def counts(ci, carry):
            nc, nr = carry
            base = ci * 16
            idx3 = (base + lane) * 3
            xj = plsc.load_gather(pos_v, [idx3])
            yj = plsc.load_gather(pos_v, [idx3 + 1])
            zj = plsc.load_gather(pos_v, [idx3 + 2])
            dx = xj - xi
            dy = yj - yi
            dz = zj - zi
            d = _sqrt16(dx * dx + dy * dy + dz * dz)
            notself = (base + lane) != i
            mr = (d <= RCR) & notself
            frad = jnp.where(mr, 0.25 * _fc16(d, RCR), 0.0)
            incr = plsc.cumsum(mr.astype(jnp.int32))
            posr = nr + incr - 1
            plsc.store_scatter(nrd, [posr], d, mask=mr)
            plsc.store_scatter(nrf, [posr], frad, mask=mr)
            ma = (d <= RCA) & notself
            fca = jnp.where(ma, _fc16(d, RCA), 0.0)
            inca = plsc.cumsum(ma.astype(jnp.int32))
            posa = nc + inca - 1
            plsc.store_scatter(nbx, [posa], xj, mask=ma)
            plsc.store_scatter(nby, [posa], yj, mask=ma)
            plsc.store_scatter(nbz, [posa], zj, mask=ma)
            plsc.store_scatter(nbd, [posa], d, mask=ma)
            plsc.store_scatter(nbf, [posa], fca, mask=ma)
            return nc + jnp.max(inca), nr + jnp.max(incr)

        nc, nr = counts

        def r_body(rj, a):
            rv = jnp.full((16,), rj, jnp.int32)
            dv = plsc.load_gather(nrd, [rv])
            fv = plsc.load_gather(nrf, [rv])
            ts = dv - shfr
            return a + fv * jnp.exp(-ETA_R * (ts * ts))

        acc_r = lax.fori_loop(0, nr, r_body, acc_in)

        npair = lax.shift_right_logical(nc * (nc - 1), 1)
        npc = lax.shift_right_logical(npair + 15, 4)

        def p_body(pc, a):
            tt = pc * 16 + lane
            tf = tt.astype(jnp.float32)
            kf = (1.0 + _sqrt16(1.0 + 8.0 * tf)) * 0.5
            kk = kf.astype(jnp.int32)
            tri = lax.shift_right_logical(kk * (kk - 1), 1)
            too_big = tt < tri
            kk = jnp.where(too_big, kk - 1, kk)
            tri = jnp.where(too_big, lax.shift_right_logical(kk * (kk - 1), 1),
                            tri)
            too_small = tt >= tri + kk
            kk = jnp.where(too_small, kk + 1, kk)
            tri = jnp.where(too_small,
                            lax.shift_right_logical(kk * (kk - 1), 1), tri)
            jj = tt - tri
            valid = tt < npair
            jj = jnp.where(valid, jj, 0)
            kk = jnp.where(valid, kk, 1)
            xj = plsc.load_gather(nbx, [jj])
            yj = plsc.load_gather(nby, [jj])
            zj = plsc.load_gather(nbz, [jj])
            dj = plsc.load_gather(nbd, [jj])
            fj = plsc.load_gather(nbf, [jj])
            xk = plsc.load_gather(nbx, [kk])
            yk = plsc.load_gather(nby, [kk])
            zk = plsc.load_gather(nbz, [kk])
            dk = plsc.load_gather(nbd, [kk])
            fk = plsc.load_gather(nbf, [kk])
            fk = jnp.where(valid, fk, 0.0)
            dots = ((xj - xi) * (xk - xi) + (yj - yi) * (yk - yi)
                    + (zj - zi) * (zk - zi))
            denom = jnp.maximum(dj * dk, 1e-10)
            c = jnp.clip(0.95 * dots / denom, -0.95, 0.95)
            s = _sqrt16(jnp.maximum(1.0 - c * c, 0.0), iters=2)
            avg = (dj + dk) * 0.5
            f2 = jnp.zeros((16,), jnp.float32)
            for sa in SHF_A:
                ta = avg - sa
                f2 = f2 + jnp.exp(-ETA_A * (ta * ta))
            f1 = jnp.zeros((16,), jnp.float32)
            for cz, sz in zip(_COS_Z, _SIN_Z):
                y = jnp.maximum((1.0 + c * cz + s * sz) * 0.5, 1e-30)
                y2 = y * y
                y4 = y2 * y2
                y8 = y4 * y4
                y14 = y8 * y4 * y2
                f1 = f1 + y14 * jnp.exp(0.1 * _ln16(y))
            return a + (2.0 * fj * fk) * (f1 * f2)

        return lax.fori_loop(0, npc, p_body, acc_r)

    acc = lax.fori_loop(0, CPW, center_body, jnp.zeros((16,), jnp.float32))
    acc_v[...] = acc
    pltpu.sync_copy(acc_v, out_hbm.at[wid])


@jax.jit
def _aev_mean_sc(positions):
    pos_flat = positions.astype(jnp.float32).reshape(3 * N)
    mesh = plsc.VectorSubcoreMesh(core_axis_name="c", subcore_axis_name="s")
    f32 = jnp.float32
    sck = pl.kernel(
        _sc_body,
        out_type=jax.ShapeDtypeStruct((NW, 16), f32),
        mesh=mesh,
        compiler_params=pltpu.CompilerParams(needs_layout_passes=False),
        scratch_types=[
            pltpu.VMEM((3 * N,), f32),
            pltpu.VMEM((CAP,), f32), pltpu.VMEM((CAP,), f32),
            pltpu.VMEM((CAP,), f32), pltpu.VMEM((CAP,), f32),
            pltpu.VMEM((CAP,), f32),
            pltpu.VMEM((CAP,), f32), pltpu.VMEM((CAP,), f32),
            pltpu.VMEM((16,), f32),
        ],
    )
    out = sck(pos_flat)
    return jnp.sum(out) * (1.0 / (N * N_FEAT))


def kernel(species, positions):
    del species
    return _aev_mean_sc(positions)


# speedup vs baseline: 1.0718x; 1.0204x over previous
import jax
import jax.numpy as jnp
import numpy as np
from jax import lax
from jax.experimental import pallas as pl
from jax.experimental.pallas import tpu as pltpu
from jax.experimental.pallas import tpu_sc as plsc

N = 160
NUM_SPECIES = 7
RCR = 5.1
RCA = 3.5
ETA_R = 19.7
SHF_R0 = 0.8
SHF_R_STEP = 0.26875
N_SHF_R = 16
ZETA = 14.1
SHF_Z = [0.19634954, 0.58904862, 0.9817477, 1.3744468, 1.7671459, 2.1598449,
         2.552544, 2.9452431]
ETA_A = 12.5
SHF_A = [0.8, 1.1375, 1.475, 1.8125, 2.15, 2.4875, 2.825, 3.1625]
NUM_PAIRS = NUM_SPECIES * (NUM_SPECIES + 1) // 2
N_FEAT = NUM_SPECIES * N_SHF_R + NUM_PAIRS * len(SHF_Z) * len(SHF_A)
PI = float(np.pi)
LN2 = float(np.log(2.0))

_COS_Z = [float(np.cos(np.float32(z))) for z in SHF_Z]
_SIN_Z = [float(np.sin(np.float32(z))) for z in SHF_Z]

NC = 2
NS = 16
NW = NC * NS
CPW = N // NW
NCHUNK = N // 16
CAP = 176


def _sqrt16(x, iters=3):
    ok = x > 1e-12
    xs = jnp.where(ok, x, 1.0)
    i = plsc.bitcast(xs, jnp.int32)
    y = plsc.bitcast(jnp.int32(0x5F3759DF) - lax.shift_right_logical(i, 1),
                     jnp.float32)
    for _ in range(iters):
        y = y * (1.5 - 0.5 * xs * y * y)
    return jnp.where(ok, xs * y, 0.0)


def _fc16(d, rc):
    x = jnp.minimum(d * (PI / (2.0 * rc)), PI / 2.0)
    u = x * x
    c = 1.0 + u * (-1.0 / 2 + u * (1.0 / 24 + u * (-1.0 / 720 + u * (
        1.0 / 40320 + u * (-1.0 / 3628800 + u * (1.0 / 479001600))))))
    return c * c


def _ln16(y):
    i = plsc.bitcast(y, jnp.int32)
    t = i.astype(jnp.float32) * (2.0 ** -23) - 127.0
    f = (i & jnp.int32(0x7FFFFF)).astype(jnp.float32) * (2.0 ** -23)
    return LN2 * (t + 0.346607 * f * (1.0 - f))


def _sc_body(pos_hbm, out_hbm,
             pos_v, nbx, nby, nbz, nbd, nbf, nrd, nrf, acc_v):
    cid = lax.axis_index("c")
    sid = lax.axis_index("s")
    wid = sid * NC + cid
    pltpu.sync_copy(pos_hbm, pos_v)
    lane = lax.iota(jnp.int32, 16)
    lane_f = lane.astype(jnp.float32)
    shfr = SHF_R0 + SHF_R_STEP * lane_f

    def center_body(t, acc_in):
        i = wid * CPW + t
        iv = jnp.full((16,), 3 * i, jnp.int32)
        xi = plsc.load_gather(pos_v, [iv])
        yi = plsc.load_gather(pos_v, [iv + 1])
        zi = plsc.load_gather(pos_v, [iv + 2])

        @plsc.parallel_loop(0, NCHUNK, 1, unroll=2,
                            carry=(jnp.int32(0), jnp.int32(0)))
        def counts(ci, carry):
            nc, nr = carry
            base = ci * 16
            idx3 = (base + lane) * 3
            xj = plsc.load_gather(pos_v, [idx3])
            yj = plsc.load_gather(pos_v, [idx3 + 1])
            zj = plsc.load_gather(pos_v, [idx3 + 2])
            dx = xj - xi
            dy = yj - yi
            dz = zj - zi
            d = _sqrt16(dx * dx + dy * dy + dz * dz)
            notself = (base + lane) != i
            mr = (d <= RCR) & notself
            frad = jnp.where(mr, 0.25 * _fc16(d, RCR), 0.0)
            incr = plsc.cumsum(mr.astype(jnp.int32))
            posr = nr + incr - 1
            plsc.store_scatter(nrd, [posr], d, mask=mr)
            plsc.store_scatter(nrf, [posr], frad, mask=mr)
            ma = (d <= RCA) & notself
            fca = jnp.where(ma, _fc16(d, RCA), 0.0)
            inca = plsc.cumsum(ma.astype(jnp.int32))
            posa = nc + inca - 1
            plsc.store_scatter(nbx, [posa], xj, mask=ma)
            plsc.store_scatter(nby, [posa], yj, mask=ma)
            plsc.store_scatter(nbz, [posa], zj, mask=ma)
            plsc.store_scatter(nbd, [posa], d, mask=ma)
            plsc.store_scatter(nbf, [posa], fca, mask=ma)
            return nc + jnp.max(inca), nr + jnp.max(incr)

        nc, nr = counts

        def r_body(rj, a):
            rv = jnp.full((16,), rj, jnp.int32)
            dv = plsc.load_gather(nrd, [rv])
            fv = plsc.load_gather(nrf, [rv])
            ts = dv - shfr
            return a + fv * jnp.exp(-ETA_R * (ts * ts))

        acc_r = lax.fori_loop(0, nr, r_body, acc_in)

        npair = lax.shift_right_logical(nc * (nc - 1), 1)
        npc = lax.shift_right_logical(npair + 15, 4)

        def p_body(pc, a):
            tt = pc * 16 + lane
            tf = tt.astype(jnp.float32)
            kf = (1.0 + _sqrt16(1.0 + 8.0 * tf)) * 0.5
            kk = kf.astype(jnp.int32)
            tri = lax.shift_right_logical(kk * (kk - 1), 1)
            too_big = tt < tri
            kk = jnp.where(too_big, kk - 1, kk)
            tri = jnp.where(too_big, lax.shift_right_logical(kk * (kk - 1), 1),
                            tri)
            too_small = tt >= tri + kk
            kk = jnp.where(too_small, kk + 1, kk)
            tri = jnp.where(too_small,
                            lax.shift_right_logical(kk * (kk - 1), 1), tri)
            jj = tt - tri
            valid = tt < npair
            jj = jnp.where(valid, jj, 0)
            kk = jnp.where(valid, kk, 1)
            xj = plsc.load_gather(nbx, [jj])
            yj = plsc.load_gather(nby, [jj])
            zj = plsc.load_gather(nbz, [jj])
            dj = plsc.load_gather(nbd, [jj])
            fj = plsc.load_gather(nbf, [jj])
            xk = plsc.load_gather(nbx, [kk])
            yk = plsc.load_gather(nby, [kk])
            zk = plsc.load_gather(nbz, [kk])
            dk = plsc.load_gather(nbd, [kk])
            fk = plsc.load_gather(nbf, [kk])
            fk = jnp.where(valid, fk, 0.0)
            dots = ((xj - xi) * (xk - xi) + (yj - yi) * (yk - yi)
                    + (zj - zi) * (zk - zi))
            denom = jnp.maximum(dj * dk, 1e-10)
            c = jnp.clip(0.95 * dots / denom, -0.95, 0.95)
            s = _sqrt16(jnp.maximum(1.0 - c * c, 0.0), iters=2)
            avg = (dj + dk) * 0.5
            f2 = jnp.zeros((16,), jnp.float32)
            for sa in SHF_A:
                ta = avg - sa
                f2 = f2 + jnp.exp(-ETA_A * (ta * ta))
            f1 = jnp.zeros((16,), jnp.float32)
            for cz, sz in zip(_COS_Z, _SIN_Z):
                y = jnp.maximum((1.0 + c * cz + s * sz) * 0.5, 1e-30)
                y2 = y * y
                y4 = y2 * y2
                y8 = y4 * y4
                y14 = y8 * y4 * y2
                f1 = f1 + y14 * jnp.exp(0.1 * _ln16(y))
            return a + (2.0 * fj * fk) * (f1 * f2)

        return lax.fori_loop(0, npc, p_body, acc_r)

    acc = lax.fori_loop(0, CPW, center_body, jnp.zeros((16,), jnp.float32))
    acc_v[...] = acc
    pltpu.sync_copy(acc_v, out_hbm.at[wid])


@jax.jit
def _aev_mean_sc(positions):
    pos_flat = positions.astype(jnp.float32).reshape(3 * N)
    mesh = plsc.VectorSubcoreMesh(core_axis_name="c", subcore_axis_name="s")
    f32 = jnp.float32
    sck = pl.kernel(
        _sc_body,
        out_type=jax.ShapeDtypeStruct((NW, 16), f32),
        mesh=mesh,
        compiler_params=pltpu.CompilerParams(needs_layout_passes=False),
        scratch_types=[
            pltpu.VMEM((3 * N,), f32),
            pltpu.VMEM((CAP,), f32), pltpu.VMEM((CAP,), f32),
            pltpu.VMEM((CAP,), f32), pltpu.VMEM((CAP,), f32),
            pltpu.VMEM((CAP,), f32),
            pltpu.VMEM((CAP,), f32), pltpu.VMEM((CAP,), f32),
            pltpu.VMEM((16,), f32),
        ],
    )
    out = sck(pos_flat)
    return jnp.sum(out) * (1.0 / (N * N_FEAT))


def kernel(species, positions):
    del species
    return _aev_mean_sc(positions)
